# Initial kernel scaffold; baseline (speedup 1.0000x reference)
import jax
import jax.numpy as jnp
from jax.experimental import pallas as pl


def kernel(context_ids, question_ids, emb_matrix):
    raise NotImplementedError("write your pallas kernel here")


# speedup vs baseline: 1.0649x; 1.0649x over previous
import functools

import jax
import jax.numpy as jnp
from jax import lax
from jax.experimental import pallas as pl
from jax.experimental.pallas import tpu as pltpu
from jax.experimental.pallas import tpu_sc as plsc

DIM = 50
PAD_DIM = 56
BATCH = 4096
CTX_LEN = 200
Q_LEN = 20
CTX_TOKENS = BATCH * CTX_LEN
Q_TOKENS = BATCH * Q_LEN
NC = 2
NS = 16
NW = NC * NS
CHUNK = 128
CTX_CHUNKS = CTX_TOKENS // (NW * CHUNK)
Q_CHUNKS = Q_TOKENS // (NW * CHUNK)


def _make_kernel():
  mesh = plsc.VectorSubcoreMesh(core_axis_name="c", subcore_axis_name="s")

  @functools.partial(
      pl.kernel,
      mesh=mesh,
      out_type=[
          jax.ShapeDtypeStruct((CTX_TOKENS, PAD_DIM), jnp.float32),
          jax.ShapeDtypeStruct((Q_TOKENS, PAD_DIM), jnp.float32),
      ],
      scratch_types=[
          pltpu.VMEM((CTX_CHUNKS, CHUNK), jnp.int32),
          pltpu.VMEM((Q_CHUNKS, CHUNK), jnp.int32),
          pltpu.VMEM((CHUNK, PAD_DIM), jnp.float32),
          pltpu.SemaphoreType.DMA,
      ],
      compiler_params=pltpu.CompilerParams(use_tc_tiling_on_sc=False),
  )
  def gather_kernel(table_hbm, ctx_idx_hbm, q_idx_hbm, t_out, j_out,
                    ctx_idx_v, q_idx_v, rows_v, sem):
    wid = lax.axis_index("s") * NC + lax.axis_index("c")
    pltpu.sync_copy(ctx_idx_hbm.at[wid], ctx_idx_v)
    pltpu.sync_copy(q_idx_hbm.at[wid], q_idx_v)

    ctx_base = wid * (CTX_CHUNKS * CHUNK)
    q_base = wid * (Q_CHUNKS * CHUNK)

    def ctx_body(i, carry):
      pltpu.async_copy(table_hbm.at[ctx_idx_v.at[i]], rows_v, sem).wait()
      pltpu.sync_copy(rows_v, t_out.at[pl.ds(ctx_base + i * CHUNK, CHUNK)])
      return carry

    lax.fori_loop(0, CTX_CHUNKS, ctx_body, 0)

    def q_body(i, carry):
      pltpu.async_copy(table_hbm.at[q_idx_v.at[i]], rows_v, sem).wait()
      pltpu.sync_copy(rows_v, j_out.at[pl.ds(q_base + i * CHUNK, CHUNK)])
      return carry

    lax.fori_loop(0, Q_CHUNKS, q_body, 0)

  return gather_kernel


_GATHER = _make_kernel()


@jax.jit
def kernel(context_ids, question_ids, emb_matrix):
  emb_padded = jnp.pad(emb_matrix, ((0, 0), (0, PAD_DIM - DIM)))
  ctx = context_ids.astype(jnp.int32).reshape(NW, CTX_CHUNKS, CHUNK)
  q = question_ids.astype(jnp.int32).reshape(NW, Q_CHUNKS, CHUNK)
  t_flat, j_flat = _GATHER(emb_padded, ctx, q)
  return (t_flat[:, :DIM].reshape(BATCH, CTX_LEN, DIM),
          j_flat[:, :DIM].reshape(BATCH, Q_LEN, DIM))


# speedup vs baseline: 2.1124x; 1.9836x over previous
import functools

import jax
import jax.numpy as jnp
from jax import lax
from jax.experimental import pallas as pl
from jax.experimental.pallas import tpu as pltpu
from jax.experimental.pallas import tpu_sc as plsc

DIM = 50
PAD_DIM = 56
TBL_DIM = 128
BATCH = 4096
CTX_LEN = 200
Q_LEN = 20
CTX_TOKENS = BATCH * CTX_LEN
Q_TOKENS = BATCH * Q_LEN
NC = 2
NS = 16
NW = NC * NS
CHUNK = 128
CTX_CHUNKS = CTX_TOKENS // (NW * CHUNK)
Q_CHUNKS = Q_TOKENS // (NW * CHUNK)


def _make_kernel():
  mesh = plsc.VectorSubcoreMesh(core_axis_name="c", subcore_axis_name="s")

  @functools.partial(
      pl.kernel,
      mesh=mesh,
      out_type=[
          jax.ShapeDtypeStruct((CTX_TOKENS, TBL_DIM), jnp.float32),
          jax.ShapeDtypeStruct((Q_TOKENS, TBL_DIM), jnp.float32),
      ],
      scratch_types=[
          pltpu.VMEM((CTX_CHUNKS, CHUNK), jnp.int32),
          pltpu.VMEM((Q_CHUNKS, CHUNK), jnp.int32),
          pltpu.VMEM((CHUNK, TBL_DIM), jnp.float32),
          pltpu.VMEM((CHUNK, TBL_DIM), jnp.float32),
          pltpu.SemaphoreType.DMA,
          pltpu.SemaphoreType.DMA,
          pltpu.SemaphoreType.DMA,
      ],
      compiler_params=pltpu.CompilerParams(use_tc_tiling_on_sc=False),
  )
  def gather_kernel(table_hbm, ctx_idx_hbm, q_idx_hbm, t_out, j_out,
                    ctx_idx_v, q_idx_v, rows0_v, rows1_v, sem_g, sem_s0,
                    sem_s1):
    wid = lax.axis_index("s") * NC + lax.axis_index("c")
    pltpu.sync_copy(ctx_idx_hbm.at[wid], ctx_idx_v)
    pltpu.sync_copy(q_idx_hbm.at[wid], q_idx_v)

    ctx_base = wid * (CTX_CHUNKS * CHUNK)
    q_base = wid * (Q_CHUNKS * CHUNK)

    def make_pair_body(idx_v, out_ref, base, n_pairs):
      def pair_body(k, carry):
        i0 = 2 * k
        dst0 = out_ref.at[pl.ds(base + i0 * CHUNK, CHUNK)]
        dst1 = out_ref.at[pl.ds(base + (i0 + 1) * CHUNK, CHUNK)]

        @pl.when(k > 0)
        def _():
          pltpu.make_async_copy(rows0_v, dst0, sem_s0).wait()

        pltpu.async_copy(table_hbm.at[idx_v.at[i0]], rows0_v, sem_g).wait()
        pltpu.async_copy(rows0_v, dst0, sem_s0)

        @pl.when(k > 0)
        def _():
          pltpu.make_async_copy(rows1_v, dst1, sem_s1).wait()

        pltpu.async_copy(table_hbm.at[idx_v.at[i0 + 1]], rows1_v, sem_g).wait()
        pltpu.async_copy(rows1_v, dst1, sem_s1)
        return carry

      lax.fori_loop(0, n_pairs, pair_body, 0)
      pltpu.make_async_copy(rows0_v, out_ref.at[pl.ds(base, CHUNK)],
                            sem_s0).wait()
      pltpu.make_async_copy(rows1_v, out_ref.at[pl.ds(base, CHUNK)],
                            sem_s1).wait()

    make_pair_body(ctx_idx_v, t_out, ctx_base, CTX_CHUNKS // 2)
    make_pair_body(q_idx_v, j_out, q_base, Q_CHUNKS // 2)

  return gather_kernel


_GATHER = _make_kernel()

_PAD_ROWS = 10000


def _pad_block(in_ref, out_ref):
  out_ref[:, :DIM] = in_ref[...]


def _pad_table(emb_matrix):
  vocab = emb_matrix.shape[0]
  return pl.pallas_call(
      _pad_block,
      grid=(vocab // _PAD_ROWS,),
      in_specs=[pl.BlockSpec((_PAD_ROWS, DIM), lambda i: (i, 0))],
      out_specs=pl.BlockSpec((_PAD_ROWS, TBL_DIM), lambda i: (i, 0)),
      out_shape=jax.ShapeDtypeStruct((vocab, TBL_DIM), jnp.float32),
  )(emb_matrix)


@jax.jit
def kernel(context_ids, question_ids, emb_matrix):
  emb_padded = _pad_table(emb_matrix)
  ctx = context_ids.astype(jnp.int32).reshape(NW, CTX_CHUNKS, CHUNK)
  q = question_ids.astype(jnp.int32).reshape(NW, Q_CHUNKS, CHUNK)
  t_flat, j_flat = _GATHER(emb_padded, ctx, q)
  return (t_flat[:, :DIM].reshape(BATCH, CTX_LEN, DIM),
          j_flat[:, :DIM].reshape(BATCH, Q_LEN, DIM))


# speedup vs baseline: 2.1662x; 1.0255x over previous
import functools

import jax
import jax.numpy as jnp
from jax import lax
from jax.experimental import pallas as pl
from jax.experimental.pallas import tpu as pltpu
from jax.experimental.pallas import tpu_sc as plsc

DIM = 50
PAD_DIM = 56
TBL_DIM = 128
BATCH = 4096
CTX_LEN = 200
Q_LEN = 20
CTX_TOKENS = BATCH * CTX_LEN
Q_TOKENS = BATCH * Q_LEN
NC = 2
NS = 16
NW = NC * NS
CHUNK = 128
CTX_CHUNKS = CTX_TOKENS // (NW * CHUNK)
Q_CHUNKS = Q_TOKENS // (NW * CHUNK)


def _make_kernel():
  mesh = plsc.VectorSubcoreMesh(core_axis_name="c", subcore_axis_name="s")

  @functools.partial(
      pl.kernel,
      mesh=mesh,
      out_type=[
          jax.ShapeDtypeStruct((CTX_TOKENS, TBL_DIM), jnp.float32),
          jax.ShapeDtypeStruct((Q_TOKENS, TBL_DIM), jnp.float32),
      ],
      scratch_types=[
          pltpu.VMEM((CTX_CHUNKS, CHUNK), jnp.int32),
          pltpu.VMEM((Q_CHUNKS, CHUNK), jnp.int32),
          pltpu.VMEM((CHUNK, TBL_DIM), jnp.float32),
          pltpu.VMEM((CHUNK, TBL_DIM), jnp.float32),
          pltpu.SemaphoreType.DMA,
          pltpu.SemaphoreType.DMA,
          pltpu.SemaphoreType.DMA,
          pltpu.SemaphoreType.DMA,
      ],
      compiler_params=pltpu.CompilerParams(use_tc_tiling_on_sc=False),
  )
  def gather_kernel(table_hbm, ctx_idx_hbm, q_idx_hbm, t_out, j_out,
                    ctx_idx_v, q_idx_v, rows0_v, rows1_v, sem_g0, sem_g1,
                    sem_s0, sem_s1):
    wid = lax.axis_index("s") * NC + lax.axis_index("c")
    pltpu.sync_copy(ctx_idx_hbm.at[wid], ctx_idx_v)
    pltpu.sync_copy(q_idx_hbm.at[wid], q_idx_v)

    ctx_base = wid * (CTX_CHUNKS * CHUNK)
    q_base = wid * (Q_CHUNKS * CHUNK)

    def make_pair_body(idx_v, out_ref, base, n_pairs):
      def pair_body(k, carry):
        i0 = 2 * k
        dst0 = out_ref.at[pl.ds(base + i0 * CHUNK, CHUNK)]
        dst1 = out_ref.at[pl.ds(base + (i0 + 1) * CHUNK, CHUNK)]

        @pl.when(k > 0)
        def _():
          pltpu.make_async_copy(rows0_v, dst0, sem_s0).wait()
          pltpu.make_async_copy(rows1_v, dst1, sem_s1).wait()

        g0 = pltpu.async_copy(table_hbm.at[idx_v.at[i0]], rows0_v, sem_g0)
        g1 = pltpu.async_copy(table_hbm.at[idx_v.at[i0 + 1]], rows1_v,
                              sem_g1)
        g0.wait()
        pltpu.async_copy(rows0_v, dst0, sem_s0)
        g1.wait()
        pltpu.async_copy(rows1_v, dst1, sem_s1)
        return carry

      lax.fori_loop(0, n_pairs, pair_body, 0)
      pltpu.make_async_copy(rows0_v, out_ref.at[pl.ds(base, CHUNK)],
                            sem_s0).wait()
      pltpu.make_async_copy(rows1_v, out_ref.at[pl.ds(base, CHUNK)],
                            sem_s1).wait()

    make_pair_body(ctx_idx_v, t_out, ctx_base, CTX_CHUNKS // 2)
    make_pair_body(q_idx_v, j_out, q_base, Q_CHUNKS // 2)

  return gather_kernel


_GATHER = _make_kernel()

_PAD_ROWS = 10000


def _pad_block(in_ref, out_ref):
  out_ref[:, :DIM] = in_ref[...]


def _pad_table(emb_matrix):
  vocab = emb_matrix.shape[0]
  return pl.pallas_call(
      _pad_block,
      grid=(vocab // _PAD_ROWS,),
      in_specs=[pl.BlockSpec((_PAD_ROWS, DIM), lambda i: (i, 0))],
      out_specs=pl.BlockSpec((_PAD_ROWS, TBL_DIM), lambda i: (i, 0)),
      out_shape=jax.ShapeDtypeStruct((vocab, TBL_DIM), jnp.float32),
  )(emb_matrix)


@jax.jit
def kernel(context_ids, question_ids, emb_matrix):
  emb_padded = _pad_table(emb_matrix)
  ctx = context_ids.astype(jnp.int32).reshape(NW, CTX_CHUNKS, CHUNK)
  q = question_ids.astype(jnp.int32).reshape(NW, Q_CHUNKS, CHUNK)
  t_flat, j_flat = _GATHER(emb_padded, ctx, q)
  return (t_flat[:, :DIM].reshape(BATCH, CTX_LEN, DIM),
          j_flat[:, :DIM].reshape(BATCH, Q_LEN, DIM))


# speedup vs baseline: 2.2251x; 1.0272x over previous
import functools

import jax
import jax.numpy as jnp
from jax import lax
from jax.experimental import pallas as pl
from jax.experimental.pallas import tpu as pltpu
from jax.experimental.pallas import tpu_sc as plsc

DIM = 50
PAD_DIM = 56
TBL_DIM = 128
BATCH = 4096
CTX_LEN = 200
Q_LEN = 20
CTX_TOKENS = BATCH * CTX_LEN
Q_TOKENS = BATCH * Q_LEN
NC = 2
NS = 16
NW = NC * NS
CHUNK = 128
CTX_CHUNKS = CTX_TOKENS // (NW * CHUNK)
Q_CHUNKS = Q_TOKENS // (NW * CHUNK)


def _make_kernel():
  mesh = plsc.VectorSubcoreMesh(core_axis_name="c", subcore_axis_name="s")

  @functools.partial(
      pl.kernel,
      mesh=mesh,
      out_type=[
          jax.ShapeDtypeStruct((CTX_TOKENS, TBL_DIM), jnp.float32),
          jax.ShapeDtypeStruct((Q_TOKENS, TBL_DIM), jnp.float32),
      ],
      scratch_types=[
          pltpu.VMEM((CTX_CHUNKS, CHUNK), jnp.int32),
          pltpu.VMEM((Q_CHUNKS, CHUNK), jnp.int32),
          pltpu.VMEM((CHUNK, TBL_DIM), jnp.float32),
          pltpu.VMEM((CHUNK, TBL_DIM), jnp.float32),
          pltpu.VMEM((CHUNK, TBL_DIM), jnp.float32),
          pltpu.VMEM((CHUNK, TBL_DIM), jnp.float32),
          pltpu.SemaphoreType.DMA,
          pltpu.SemaphoreType.DMA,
          pltpu.SemaphoreType.DMA,
          pltpu.SemaphoreType.DMA,
          pltpu.SemaphoreType.DMA,
          pltpu.SemaphoreType.DMA,
          pltpu.SemaphoreType.DMA,
          pltpu.SemaphoreType.DMA,
      ],
      compiler_params=pltpu.CompilerParams(use_tc_tiling_on_sc=False),
  )
  def gather_kernel(table_hbm, ctx_idx_hbm, q_idx_hbm, t_out, j_out,
                    ctx_idx_v, q_idx_v, rows0_v, rows1_v, rows2_v, rows3_v,
                    sem_g0, sem_g1, sem_g2, sem_g3,
                    sem_s0, sem_s1, sem_s2, sem_s3):
    wid = lax.axis_index("s") * NC + lax.axis_index("c")
    pltpu.sync_copy(ctx_idx_hbm.at[wid], ctx_idx_v)
    pltpu.sync_copy(q_idx_hbm.at[wid], q_idx_v)

    ctx_base = wid * (CTX_CHUNKS * CHUNK)
    q_base = wid * (Q_CHUNKS * CHUNK)

    rows = (rows0_v, rows1_v, rows2_v, rows3_v)
    sem_g = (sem_g0, sem_g1, sem_g2, sem_g3)
    sem_s = (sem_s0, sem_s1, sem_s2, sem_s3)
    nbuf = 4

    def make_ring_body(idx_v, out_ref, base, n_groups):
      def group_body(k, carry):
        i0 = nbuf * k
        dsts = [out_ref.at[pl.ds(base + (i0 + b) * CHUNK, CHUNK)]
                for b in range(nbuf)]

        @pl.when(k > 0)
        def _():
          for b in range(nbuf):
            pltpu.make_async_copy(rows[b], dsts[b], sem_s[b]).wait()

        gathers = [
            pltpu.async_copy(table_hbm.at[idx_v.at[i0 + b]], rows[b],
                             sem_g[b])
            for b in range(nbuf)
        ]
        for b in range(nbuf):
          gathers[b].wait()
          pltpu.async_copy(rows[b], dsts[b], sem_s[b])
        return carry

      lax.fori_loop(0, n_groups, group_body, 0)
      for b in range(nbuf):
        pltpu.make_async_copy(rows[b], out_ref.at[pl.ds(base, CHUNK)],
                              sem_s[b]).wait()

    make_ring_body(ctx_idx_v, t_out, ctx_base, CTX_CHUNKS // nbuf)
    make_ring_body(q_idx_v, j_out, q_base, Q_CHUNKS // nbuf)

  return gather_kernel


_GATHER = _make_kernel()

_PAD_ROWS = 10000


def _pad_block(in_ref, out_ref):
  out_ref[:, :DIM] = in_ref[...]


def _pad_table(emb_matrix):
  vocab = emb_matrix.shape[0]
  return pl.pallas_call(
      _pad_block,
      grid=(vocab // _PAD_ROWS,),
      in_specs=[pl.BlockSpec((_PAD_ROWS, DIM), lambda i: (i, 0))],
      out_specs=pl.BlockSpec((_PAD_ROWS, TBL_DIM), lambda i: (i, 0)),
      out_shape=jax.ShapeDtypeStruct((vocab, TBL_DIM), jnp.float32),
  )(emb_matrix)


@jax.jit
def kernel(context_ids, question_ids, emb_matrix):
  emb_padded = _pad_table(emb_matrix)
  ctx = context_ids.astype(jnp.int32).reshape(NW, CTX_CHUNKS, CHUNK)
  q = question_ids.astype(jnp.int32).reshape(NW, Q_CHUNKS, CHUNK)
  t_flat, j_flat = _GATHER(emb_padded, ctx, q)
  return (t_flat[:, :DIM].reshape(BATCH, CTX_LEN, DIM),
          j_flat[:, :DIM].reshape(BATCH, Q_LEN, DIM))
